# Initial kernel scaffold; baseline (speedup 1.0000x reference)
import jax
import jax.numpy as jnp
from jax.experimental import pallas as pl


def kernel(x, feature_embedding, W1, b1, W2, b2, W3, b3):
    raise NotImplementedError("write your pallas kernel here")


# speedup vs baseline: 1.1662x; 1.1662x over previous
import jax
import jax.numpy as jnp
from jax.experimental import pallas as pl
from jax.experimental.pallas import tpu as pltpu

B = 8
N = 8192
K = 2048
CHUNK = 2048
NC = N // CHUNK


def _chowder_kernel(x_ref, fe_ref, w1_ref, b1_ref, w2_ref, b2_ref,
                    w3_ref, b3_ref, out_ref, ext_ref):
    b = pl.program_id(0)
    c = pl.program_id(1)

    @pl.when(c == 0)
    def _init():
        ext_ref[b, 0] = jnp.inf
        ext_ref[b, 1] = jnp.inf
        ext_ref[b, 2] = -jnp.inf
        ext_ref[b, 3] = -jnp.inf

    vals = jnp.dot(x_ref[0], fe_ref[...],
                   preferred_element_type=jnp.float32)

    m1 = jnp.max(vals)
    mcnt = jnp.sum(jnp.where(vals == m1, 1.0, 0.0))
    m2 = jnp.where(mcnt >= 2.0, m1,
                   jnp.max(jnp.where(vals == m1, -jnp.inf, vals)))
    n1 = jnp.min(vals)
    ncnt = jnp.sum(jnp.where(vals == n1, 1.0, 0.0))
    n2 = jnp.where(ncnt >= 2.0, n1,
                   jnp.min(jnp.where(vals == n1, jnp.inf, vals)))

    a1 = ext_ref[b, 3]
    a2 = ext_ref[b, 2]
    ext_ref[b, 3] = jnp.maximum(a1, m1)
    ext_ref[b, 2] = jnp.maximum(jnp.minimum(a1, m1), jnp.maximum(a2, m2))
    s1 = ext_ref[b, 0]
    s2 = ext_ref[b, 1]
    ext_ref[b, 0] = jnp.minimum(s1, n1)
    ext_ref[b, 1] = jnp.minimum(jnp.maximum(s1, n1), jnp.minimum(s2, n2))

    @pl.when((b == B - 1) & (c == NC - 1))
    def _head():
        mm = jnp.stack(
            [jnp.stack([ext_ref[i, j] for j in range(4)]) for i in range(B)])
        h = jax.nn.sigmoid(
            jnp.dot(mm, w1_ref[...], preferred_element_type=jnp.float32)
            + b1_ref[...])
        h = jax.nn.sigmoid(
            jnp.dot(h, w2_ref[...], preferred_element_type=jnp.float32)
            + b2_ref[...])
        logits = (jnp.dot(h, w3_ref[...], preferred_element_type=jnp.float32)
                  + b3_ref[...])
        z = logits - jnp.max(logits, axis=1, keepdims=True)
        e = jnp.exp(z)
        out_ref[...] = e / jnp.sum(e, axis=1, keepdims=True)


def kernel(x, feature_embedding, W1, b1, W2, b2, W3, b3):
    fe = feature_embedding.reshape(K, 1)
    w1t = W1.T
    b1r = b1.reshape(1, -1)
    w2t = W2.T
    b2r = b2.reshape(1, -1)
    w3t = W3.T
    b3r = b3.reshape(1, -1)

    grid = (B, NC)
    out = pl.pallas_call(
        _chowder_kernel,
        grid=grid,
        in_specs=[
            pl.BlockSpec((1, CHUNK, K), lambda b, c: (b, c, 0)),
            pl.BlockSpec((K, 1), lambda b, c: (0, 0)),
            pl.BlockSpec((4, 200), lambda b, c: (0, 0)),
            pl.BlockSpec((1, 200), lambda b, c: (0, 0)),
            pl.BlockSpec((200, 100), lambda b, c: (0, 0)),
            pl.BlockSpec((1, 100), lambda b, c: (0, 0)),
            pl.BlockSpec((100, 1), lambda b, c: (0, 0)),
            pl.BlockSpec((1, 1), lambda b, c: (0, 0)),
        ],
        out_specs=pl.BlockSpec((B, 1), lambda b, c: (0, 0)),
        out_shape=jax.ShapeDtypeStruct((B, 1), jnp.float32),
        scratch_shapes=[pltpu.SMEM((B, 4), jnp.float32)],
    )(x, fe, w1t, b1r, w2t, b2r, w3t, b3r)
    return out


# speedup vs baseline: 1.2247x; 1.0502x over previous
import jax
import jax.numpy as jnp
from jax.experimental import pallas as pl
from jax.experimental.pallas import tpu as pltpu

B = 8
N = 8192
K = 2048
CHUNK = 2048
NC = N // CHUNK


def _chowder_kernel(x_ref, fe_ref, w1_ref, b1_ref, w2_ref, b2_ref,
                    w3_ref, b3_ref, out_ref, ext_ref):
    b = pl.program_id(0)
    c = pl.program_id(1)

    @pl.when(c == 0)
    def _init():
        ext_ref[b, 0] = jnp.inf
        ext_ref[b, 1] = jnp.inf
        ext_ref[b, 2] = -jnp.inf
        ext_ref[b, 3] = -jnp.inf

    vals = jax.lax.dot_general(
        fe_ref[...], x_ref[0], (((1,), (1,)), ((), ())),
        preferred_element_type=jnp.float32)

    m1 = jnp.max(vals)
    mcnt = jnp.sum(jnp.where(vals == m1, 1.0, 0.0))
    m2 = jnp.where(mcnt >= 2.0, m1,
                   jnp.max(jnp.where(vals == m1, -jnp.inf, vals)))
    n1 = jnp.min(vals)
    ncnt = jnp.sum(jnp.where(vals == n1, 1.0, 0.0))
    n2 = jnp.where(ncnt >= 2.0, n1,
                   jnp.min(jnp.where(vals == n1, jnp.inf, vals)))

    a1 = ext_ref[b, 3]
    a2 = ext_ref[b, 2]
    ext_ref[b, 3] = jnp.maximum(a1, m1)
    ext_ref[b, 2] = jnp.maximum(jnp.minimum(a1, m1), jnp.maximum(a2, m2))
    s1 = ext_ref[b, 0]
    s2 = ext_ref[b, 1]
    ext_ref[b, 0] = jnp.minimum(s1, n1)
    ext_ref[b, 1] = jnp.minimum(jnp.maximum(s1, n1), jnp.minimum(s2, n2))

    @pl.when((b == B - 1) & (c == NC - 1))
    def _head():
        mm = jnp.stack(
            [jnp.stack([ext_ref[i, j] for j in range(4)]) for i in range(B)])
        h = jax.nn.sigmoid(
            jnp.dot(mm, w1_ref[...], preferred_element_type=jnp.float32)
            + b1_ref[...])
        h = jax.nn.sigmoid(
            jnp.dot(h, w2_ref[...], preferred_element_type=jnp.float32)
            + b2_ref[...])
        logits = (jnp.dot(h, w3_ref[...], preferred_element_type=jnp.float32)
                  + b3_ref[...])
        z = logits - jnp.max(logits, axis=1, keepdims=True)
        e = jnp.exp(z)
        out_ref[...] = e / jnp.sum(e, axis=1, keepdims=True)


def kernel(x, feature_embedding, W1, b1, W2, b2, W3, b3):
    fe = feature_embedding.reshape(1, K)
    w1t = W1.T
    b1r = b1.reshape(1, -1)
    w2t = W2.T
    b2r = b2.reshape(1, -1)
    w3t = W3.T
    b3r = b3.reshape(1, -1)

    grid = (B, NC)
    out = pl.pallas_call(
        _chowder_kernel,
        grid=grid,
        in_specs=[
            pl.BlockSpec((1, CHUNK, K), lambda b, c: (b, c, 0)),
            pl.BlockSpec((1, K), lambda b, c: (0, 0)),
            pl.BlockSpec((4, 200), lambda b, c: (0, 0)),
            pl.BlockSpec((1, 200), lambda b, c: (0, 0)),
            pl.BlockSpec((200, 100), lambda b, c: (0, 0)),
            pl.BlockSpec((1, 100), lambda b, c: (0, 0)),
            pl.BlockSpec((100, 1), lambda b, c: (0, 0)),
            pl.BlockSpec((1, 1), lambda b, c: (0, 0)),
        ],
        out_specs=pl.BlockSpec((B, 1), lambda b, c: (0, 0)),
        out_shape=jax.ShapeDtypeStruct((B, 1), jnp.float32),
        scratch_shapes=[pltpu.SMEM((B, 4), jnp.float32)],
    )(x, fe, w1t, b1r, w2t, b2r, w3t, b3r)
    return out
